# baseline (speedup 1.0000x reference)
import functools

import jax
import jax.numpy as jnp
import numpy as np
from jax import lax
from jax.experimental import pallas as pl
from jax.experimental.pallas import tpu as pltpu
from jax.experimental.pallas import tpu_sc as plsc

H = 8
D_HEAD = 16
HID = 128
NFB = 64
N_PT = 50000
N_PL = 10000
E1 = 50000
E2 = 160000

NC = 2
NS = 16
NW = NC * NS
CHUNK = 128

R = 1024
DW = 144
OWN = 5120
ACC = 5128

N_PTP = 50176
N_PLP = 10240
E1P = 65536
E2P = 163840


def _pad_rows(x, n):
    return jnp.pad(x, ((0, n - x.shape[0]),) + ((0, 0),) * (x.ndim - 1))


def _ln(x, g, b):
    mu = jnp.mean(x, axis=-1, keepdims=True)
    xc = x - mu
    v = jnp.mean(xc * xc, axis=-1, keepdims=True)
    return xc / jnp.sqrt(v + 1e-5) * g + b


def _embed_pt_body(mag, ct, freqs, w1, b1, ln1g, ln1b, w2, b2, ctab, olng,
                   olnb, ow, ob, o):
    xf = mag[...] * freqs[...] * (2.0 * np.pi)
    h = (jnp.cos(xf) @ w1[0:NFB, :] + jnp.sin(xf) @ w1[NFB:2 * NFB, :]
         + mag[...] * w1[2 * NFB:2 * NFB + 1, :] + b1[...])
    h = _ln(h, ln1g[...], ln1b[...])
    h = jax.nn.relu(h)
    x = h @ w2[...] + b2[...]
    oh = (ct[...] == lax.broadcasted_iota(jnp.int32, (R, ctab.shape[0]), 1))
    x = x + oh.astype(jnp.float32) @ ctab[...]
    x = _ln(x, olng[...], olnb[...])
    o[...] = jax.nn.relu(x) @ ow[...] + ob[...]


def _embed_pt(mag, ct, p):
    grid = N_PTP // R
    full = lambda a: pl.BlockSpec(a.shape, lambda i: (0,) * a.ndim)
    ctab = (p['type_pt_tab'][:, None, :] + p['side_pt_tab'][None, :, :]
            ).reshape(-1, HID)
    f = p['x_pt']
    args = (mag, ct, f['freqs'].reshape(1, NFB),
            f['w1'].reshape(2 * NFB + 1, HID),
            f['b1'].reshape(1, HID), f['ln1_g'].reshape(1, HID),
            f['ln1_b'].reshape(1, HID), f['w2'].reshape(HID, HID),
            f['b2'].reshape(1, HID), ctab, f['out_ln_g'].reshape(1, HID),
            f['out_ln_b'].reshape(1, HID), f['out_w'],
            f['out_b'].reshape(1, HID))
    specs = ([pl.BlockSpec((R, 1), lambda i: (i, 0)),
              pl.BlockSpec((R, 1), lambda i: (i, 0))]
             + [full(a) for a in args[2:]])
    return pl.pallas_call(
        _embed_pt_body, grid=(grid,), in_specs=specs,
        out_specs=pl.BlockSpec((R, HID), lambda i: (i, 0)),
        out_shape=jax.ShapeDtypeStruct((N_PTP, HID), jnp.float32),
    )(*args)


def _embed_pl_body(ct, ctab, olng, olnb, ow, ob, o):
    oh = (ct[...] == lax.broadcasted_iota(jnp.int32, (R, ctab.shape[0]), 1))
    x = oh.astype(jnp.float32) @ ctab[...]
    x = _ln(x, olng[...], olnb[...])
    o[...] = jax.nn.relu(x) @ ow[...] + ob[...]


def _embed_pl(ct, p):
    grid = N_PLP // R
    full = lambda a: pl.BlockSpec(a.shape, lambda i: (0,) * a.ndim)
    ctab = (p['type_pl_tab'][:, None, :] + p['int_pl_tab'][None, :, :]
            ).reshape(-1, HID)
    f = p['x_pl']
    args = (ct, ctab, f['out_ln_g'].reshape(1, HID),
            f['out_ln_b'].reshape(1, HID), f['out_w'],
            f['out_b'].reshape(1, HID))
    specs = ([pl.BlockSpec((R, 1), lambda i: (i, 0))]
             + [full(a) for a in args[1:]])
    return pl.pallas_call(
        _embed_pl_body, grid=(grid,), in_specs=specs,
        out_specs=pl.BlockSpec((R, HID), lambda i: (i, 0)),
        out_shape=jax.ShapeDtypeStruct((N_PLP, HID), jnp.float32),
    )(*args)


def _wrap_pi(a):
    w = a + np.pi
    w = w - (2.0 * np.pi) * jnp.floor(w / (2.0 * np.pi))
    return w - np.pi


def _edge_feat_body(has_cat, *refs):
    if has_cat:
        (sg, dg, tcat, cattab, freqs, w1, b1, ln1g, ln1b, w2, b2, olng, olnb,
         ow, ob, lnrg, lnrb, wkr, wvr, bvr, o) = refs
    else:
        (sg, dg, freqs, w1, b1, ln1g, ln1b, w2, b2, olng, olnb,
         ow, ob, lnrg, lnrb, wkr, wvr, bvr, o) = refs
    sx, sy, so = sg[:, 0:1], sg[:, 1:2], sg[:, 2:3]
    dx, dy, do_ = dg[:, 0:1], dg[:, 1:2], dg[:, 2:3]
    relx = sx - dx
    rely = sy - dy
    norm = jnp.sqrt(relx * relx + rely * rely)
    c = jnp.cos(do_)
    s = jnp.sin(do_)
    ang = jnp.arctan2(c * rely - s * relx, c * relx + s * rely)
    wrp = _wrap_pi(so - do_)
    rcols = (norm, ang, wrp)
    acc = None
    for d in range(3):
        rd = rcols[d]
        xf = rd * freqs[d:d + 1, :] * (2.0 * np.pi)
        h = (jnp.cos(xf) @ w1[d, 0:NFB, :] + jnp.sin(xf) @ w1[d, NFB:2 * NFB, :]
             + rd * w1[d, 2 * NFB:2 * NFB + 1, :] + b1[d:d + 1, :])
        h = _ln(h, ln1g[d:d + 1, :], ln1b[d:d + 1, :])
        h = jax.nn.relu(h)
        g = h @ w2[d] + b2[d:d + 1, :]
        acc = g if acc is None else acc + g
    if has_cat:
        oh = (tcat[...] == lax.broadcasted_iota(jnp.int32,
                                                (R, cattab.shape[0]), 1))
        acc = acc + oh.astype(jnp.float32) @ cattab[...]
    x = _ln(acc, olng[...], olnb[...])
    re = jax.nn.relu(x) @ ow[...] + ob[...]
    rn = _ln(re, lnrg[...], lnrb[...])
    o[...] = jnp.concatenate([rn @ wkr[...], rn @ wvr[...] + bvr[...]], axis=1)


def _edge_feat(sgeo, dgeo, tcat, fp, ap, cattab, ep):
    grid = ep // R
    full = lambda a: pl.BlockSpec(a.shape, lambda i: (0,) * a.ndim)
    has_cat = tcat is not None
    base = (fp['freqs'], fp['w1'], fp['b1'], fp['ln1_g'], fp['ln1_b'],
            fp['w2'], fp['b2'], fp['out_ln_g'].reshape(1, HID),
            fp['out_ln_b'].reshape(1, HID), fp['out_w'],
            fp['out_b'].reshape(1, HID), ap['ln_r_g'].reshape(1, HID),
            ap['ln_r_b'].reshape(1, HID), ap['wkr'], ap['wvr'],
            ap['bvr'].reshape(1, HID))
    if has_cat:
        args = (sgeo, dgeo, tcat, cattab) + base
        specs = [pl.BlockSpec((R, 16), lambda i: (i, 0)),
                 pl.BlockSpec((R, 16), lambda i: (i, 0)),
                 pl.BlockSpec((R, 1), lambda i: (i, 0)), full(cattab)]
    else:
        args = (sgeo, dgeo) + base
        specs = [pl.BlockSpec((R, 16), lambda i: (i, 0)),
                 pl.BlockSpec((R, 16), lambda i: (i, 0))]
    specs += [full(a) for a in base]
    return pl.pallas_call(
        functools.partial(_edge_feat_body, has_cat), grid=(grid,),
        in_specs=specs,
        out_specs=pl.BlockSpec((R, 2 * HID), lambda i: (i, 0)),
        out_shape=jax.ShapeDtypeStruct((ep, 2 * HID), jnp.float32),
    )(*args)


def _proj_kv_body(x, lng, lnb, wk, wv, bv, o):
    xs = _ln(x[...], lng[...], lnb[...])
    o[...] = jnp.concatenate([xs @ wk[...], xs @ wv[...] + bv[...]], axis=1)


def _proj_kv(x, lng, lnb, wk, wv, bv):
    n = x.shape[0]
    full = lambda a: pl.BlockSpec(a.shape, lambda i: (0,) * a.ndim)
    args = (x, lng.reshape(1, HID), lnb.reshape(1, HID), wk, wv,
            bv.reshape(1, HID))
    return pl.pallas_call(
        _proj_kv_body, grid=(n // R,),
        in_specs=[pl.BlockSpec((R, HID), lambda i: (i, 0))] +
                 [full(a) for a in args[1:]],
        out_specs=pl.BlockSpec((R, 2 * HID), lambda i: (i, 0)),
        out_shape=jax.ShapeDtypeStruct((n, 2 * HID), jnp.float32),
    )(*args)


def _proj_q_body(x, lng, lnb, wq, bq, o):
    xd = _ln(x[...], lng[...], lnb[...])
    o[...] = xd @ wq[...] + bq[...]


def _proj_q(x, lng, lnb, wq, bq):
    n = x.shape[0]
    full = lambda a: pl.BlockSpec(a.shape, lambda i: (0,) * a.ndim)
    args = (x, lng.reshape(1, HID), lnb.reshape(1, HID), wq,
            bq.reshape(1, HID))
    return pl.pallas_call(
        _proj_q_body, grid=(n // R,),
        in_specs=[pl.BlockSpec((R, HID), lambda i: (i, 0))] +
                 [full(a) for a in args[1:]],
        out_specs=pl.BlockSpec((R, HID), lambda i: (i, 0)),
        out_shape=jax.ShapeDtypeStruct((n, HID), jnp.float32),
    )(*args)


def _edge_attn_body(kvg, krvr, qg, pool, expand, o):
    ke = kvg[:, 0:HID] + krvr[:, 0:HID]
    ve = kvg[:, HID:2 * HID] + krvr[:, HID:2 * HID]
    sim = ((qg[...] * ke) @ pool[...]) * (1.0 / np.sqrt(D_HEAD))
    e = jnp.exp(sim)
    wv = ve * (e @ expand[...])
    ez = jnp.concatenate([e, jnp.zeros((e.shape[0], DW - HID - H), e.dtype)],
                         axis=1)
    o[...] = jnp.concatenate([wv, ez], axis=1)


def _edge_attn(kvg, krvr, qg, ep):
    pool = np.zeros((HID, H), np.float32)
    for h in range(H):
        pool[h * D_HEAD:(h + 1) * D_HEAD, h] = 1.0
    expand = jnp.asarray(np.ascontiguousarray(pool.T))
    pool = jnp.asarray(pool)
    full = lambda a: pl.BlockSpec(a.shape, lambda i: (0,) * a.ndim)
    return pl.pallas_call(
        _edge_attn_body, grid=(ep // R,),
        in_specs=[pl.BlockSpec((R, 2 * HID), lambda i: (i, 0)),
                  pl.BlockSpec((R, 2 * HID), lambda i: (i, 0)),
                  pl.BlockSpec((R, HID), lambda i: (i, 0)),
                  full(pool), full(expand)],
        out_specs=pl.BlockSpec((R, DW), lambda i: (i, 0)),
        out_shape=jax.ShapeDtypeStruct((ep, DW), jnp.float32),
    )(kvg, krvr, qg, pool, expand)


def _post_body(p0, xin, exp16, lng, lnb, wgt, wgb, bg, ws,
               bs, wo, bo, lnfg, lnfb, w1, w2, b2, o):
    ps = p0[...]
    num = ps[:, 0:HID]
    den = ps[:, HID:HID + 16] @ exp16[...]
    agg = num / (den + 1e-16)
    xd = _ln(xin[...], lng[...], lnb[...])
    g = jax.nn.sigmoid(xd @ wgt[...] + agg @ wgb[...] + bg[...])
    agg = agg + g * ((xd @ ws[...] + bs[...]) - agg)
    x = xin[...] + agg @ wo[...] + bo[...]
    hh = _ln(x, lnfg[...], lnfb[...])
    o[...] = x + jax.nn.relu(hh @ w1[...]) @ w2[...] + b2[...]


def _post(parts, xin, ap, ln_key):
    exp16 = np.zeros((16, HID), np.float32)
    for h in range(H):
        exp16[h, h * D_HEAD:(h + 1) * D_HEAD] = 1.0
    exp16 = jnp.asarray(exp16)
    full = lambda a: pl.BlockSpec(a.shape, lambda i: (0,) * a.ndim)
    consts = (exp16, ap[ln_key + '_g'].reshape(1, HID),
              ap[ln_key + '_b'].reshape(1, HID), ap['wg'][:HID, :],
              ap['wg'][HID:, :], ap['bg'].reshape(1, HID), ap['ws'],
              ap['bs'].reshape(1, HID), ap['wo'],
              ap['bo'].reshape(1, HID), ap['ln_ff_g'].reshape(1, HID),
              ap['ln_ff_b'].reshape(1, HID), ap['w1'], ap['w2'],
              ap['b2'].reshape(1, HID))
    return pl.pallas_call(
        _post_body, grid=(N_PLP // R,),
        in_specs=[pl.BlockSpec((R, DW), lambda i: (i, 0)),
                  pl.BlockSpec((R, HID), lambda i: (i, 0))] +
                 [full(a) for a in consts],
        out_specs=pl.BlockSpec((R, HID), lambda i: (i, 0)),
        out_shape=jax.ShapeDtypeStruct((N_PLP, HID), jnp.float32),
    )(parts, xin, *consts)


def _sc_gather2(ta, ia, tb, ib, ep, linear=False):
    da = ta.shape[1]
    db = tb.shape[1]
    ch = ep // (NW * CHUNK)
    ia = ia.reshape(NW, ch, CHUNK)
    ib = ib.reshape(NW, ch, CHUNK)
    mesh = plsc.VectorSubcoreMesh(core_axis_name="c", subcore_axis_name="s")

    @functools.partial(
        pl.kernel, mesh=mesh,
        out_type=(jax.ShapeDtypeStruct((ep, da), jnp.float32),
                  jax.ShapeDtypeStruct((ep, db), jnp.float32)),
        scratch_types=[pltpu.VMEM((ch, CHUNK), jnp.int32),
                       pltpu.VMEM((ch, CHUNK), jnp.int32),
                       pltpu.VMEM((CHUNK, da), jnp.float32),
                       pltpu.VMEM((CHUNK, db), jnp.float32),
                       pltpu.SemaphoreType.DMA,
                       pltpu.SemaphoreType.DMA],
        compiler_params=pltpu.CompilerParams(use_tc_tiling_on_sc=not linear),
    )
    def k(ta_h, ia_h, tb_h, ib_h, oa_h, ob_h, ia_v, ib_v, ra_v, rb_v, sa, sb):
        wid = lax.axis_index("s") * NC + lax.axis_index("c")
        pltpu.sync_copy(ia_h.at[wid], ia_v)
        pltpu.sync_copy(ib_h.at[wid], ib_v)

        def body(j, carry):
            base = wid * ch * CHUNK + j * CHUNK
            cpa = pltpu.async_copy(ta_h.at[ia_v.at[j]], ra_v, sa)
            cpb = pltpu.async_copy(tb_h.at[ib_v.at[j]], rb_v, sb)
            cpa.wait()
            cpb.wait()
            pltpu.sync_copy(ra_v, oa_h.at[pl.ds(base, CHUNK)])
            pltpu.sync_copy(rb_v, ob_h.at[pl.ds(base, CHUNK)])
            return carry

        lax.fori_loop(0, ch, body, 0)

    return k(ta, ia, tb, ib)


def _sc_scatter_add(vals, idx, ep):
    chw = ep // (NS * CHUNK)
    idx = idx.reshape(NS, chw, CHUNK)
    zb = jnp.zeros((64, DW), jnp.float32)
    mesh = plsc.VectorSubcoreMesh(core_axis_name="c", subcore_axis_name="s")

    @functools.partial(
        pl.kernel, mesh=mesh,
        out_type=jax.ShapeDtypeStruct((N_PLP, DW), jnp.float32),
        scratch_types=[pltpu.VMEM((chw, CHUNK), jnp.int32),
                       pltpu.VMEM((chw, CHUNK), jnp.int32),
                       pltpu.VMEM((CHUNK, DW), jnp.float32),
                       pltpu.VMEM_SHARED((ACC, DW), jnp.float32)],
        compiler_params=pltpu.CompilerParams(use_tc_tiling_on_sc=False),
    )
    def k(v_h, i_h, z_h, o_h, i_v, ia_v, v_v, acc):
        cid = lax.axis_index("c")
        sid = lax.axis_index("s")
        own0 = cid * OWN

        def zbody(t, carry):
            pltpu.sync_copy(z_h, acc.at[pl.ds(sid * (OWN // NS) + t * 64, 64)])
            return carry

        lax.fori_loop(0, OWN // NS // 64, zbody, 0)

        @pl.when(sid == 0)
        def _zero_tail():
            pltpu.sync_copy(z_h.at[pl.ds(0, ACC - OWN)],
                            acc.at[pl.ds(OWN, ACC - OWN)])

        pltpu.sync_copy(i_h.at[sid], i_v)

        def adj(j, carry):
            for l in range(CHUNK // 16):
                v = i_v[j, pl.ds(l * 16, 16)]
                ok = (v >= own0) & (v < own0 + OWN)
                ia_v[j, pl.ds(l * 16, 16)] = jnp.where(ok, v - own0, OWN)
            return carry

        lax.fori_loop(0, chw, adj, 0)
        plsc.subcore_barrier()

        def body(j, carry):
            base = sid * chw * CHUNK + j * CHUNK
            pltpu.sync_copy(v_h.at[pl.ds(base, CHUNK)], v_v)
            pltpu.sync_copy(v_v, acc.at[ia_v.at[j]], add=True)
            return carry

        lax.fori_loop(0, chw, body, 0)
        plsc.subcore_barrier()
        rpt = OWN // NS
        pltpu.sync_copy(acc.at[pl.ds(sid * rpt, rpt)],
                        o_h.at[pl.ds(own0 + sid * rpt, rpt)])

    return k(vals, idx, zb)


def kernel(pos_pt, orient_pt, magnitude_pt, type_pt, side_pt, pos_pl,
           orient_pl, type_pl, int_pl, edge_index_pt2pl, edge_index_pl2pl,
           type_pl2pl, params):
    p = params
    f32 = jnp.float32

    mag = _pad_rows(magnitude_pt[:, None].astype(f32), N_PTP)
    ct_pt = _pad_rows((type_pt.astype(jnp.int32) * 3
                       + side_pt.astype(jnp.int32))[:, None], N_PTP)
    ct_pl = _pad_rows((type_pl.astype(jnp.int32) * 3
                       + int_pl.astype(jnp.int32))[:, None], N_PLP)
    pt_params = {'type_pt_tab': p['type_pt'], 'side_pt_tab': p['side_pt'],
                 'x_pt': p['x_pt']}
    pl_params = {'type_pl_tab': p['type_pl'], 'int_pl_tab': p['int_pl'],
                 'x_pl': p['x_pl']}
    x_pt = _embed_pt(mag, ct_pt, pt_params)
    x_pl = _embed_pl(ct_pl, pl_params)

    geo_pt = jnp.concatenate(
        [pos_pt.astype(f32), orient_pt[:, None].astype(f32),
         jnp.zeros((N_PT, 13), f32)], axis=1)
    geo_pl = jnp.concatenate(
        [pos_pl.astype(f32), orient_pl[:, None].astype(f32),
         jnp.zeros((N_PL, 13), f32)], axis=1)
    s1 = edge_index_pt2pl[0].astype(jnp.int32)
    d1 = edge_index_pt2pl[1].astype(jnp.int32)
    s2 = edge_index_pl2pl[0].astype(jnp.int32)
    d2 = edge_index_pl2pl[1].astype(jnp.int32)
    pad_i = lambda a, n, v: jnp.pad(a, (0, n - a.shape[0]), constant_values=v)
    s1p = pad_i(s1, E1P, 0)
    d1p = pad_i(d1, E1P, 0)
    d1s = pad_i(d1, E1P, N_PLP - 1)
    s2p = pad_i(s2, E2P, 0)
    d2p = pad_i(d2, E2P, 0)
    d2s = pad_i(d2, E2P, N_PLP - 1)
    tc2 = _pad_rows(type_pl2pl.astype(jnp.int32)[:, None], E2P)

    sg1, dg1 = _sc_gather2(geo_pt, s1p, geo_pl, d1p, E1P, linear=True)
    sg2, dg2 = _sc_gather2(geo_pl, s2p, geo_pl, d2p, E2P, linear=True)

    a1 = p['pt2pl'][0]
    a2 = p['pl2pl'][0]
    krvr1 = _edge_feat(sg1, dg1, None, p['r_pt2pl'], a1, None, E1P)
    krvr2 = _edge_feat(sg2, dg2, tc2, p['r_pl2pl'], a2, p['type_pl2pl'], E2P)

    kv1 = _proj_kv(x_pt, a1['ln_s_g'], a1['ln_s_b'], a1['wk'], a1['wv'],
                   a1['bv'])
    q1 = _proj_q(x_pl, a1['ln_d_g'], a1['ln_d_b'], a1['wq'], a1['bq'])
    kvg1, qg1 = _sc_gather2(kv1, s1p, q1, d1p, E1P)
    ew1 = _edge_attn(kvg1, krvr1, qg1, E1P)
    parts1 = _sc_scatter_add(ew1, d1s, E1P)
    x_pl = _post(parts1, x_pl, a1, 'ln_d')

    kv2 = _proj_kv(x_pl, a2['ln_s_g'], a2['ln_s_b'], a2['wk'], a2['wv'],
                   a2['bv'])
    q2 = _proj_q(x_pl, a2['ln_s_g'], a2['ln_s_b'], a2['wq'], a2['bq'])
    kvg2, qg2 = _sc_gather2(kv2, s2p, q2, d2p, E2P)
    ew2 = _edge_attn(kvg2, krvr2, qg2, E2P)
    parts2 = _sc_scatter_add(ew2, d2s, E2P)
    x_pl = _post(parts2, x_pl, a2, 'ln_s')

    return x_pl[:N_PL]


# speedup vs baseline: 1.0046x; 1.0046x over previous
import functools

import jax
import jax.numpy as jnp
import numpy as np
from jax import lax
from jax.experimental import pallas as pl
from jax.experimental.pallas import tpu as pltpu
from jax.experimental.pallas import tpu_sc as plsc

H = 8
D_HEAD = 16
HID = 128
NFB = 64
N_PT = 50000
N_PL = 10000
E1 = 50000
E2 = 160000

NC = 2
NS = 16
NW = NC * NS
CHUNK = 128

R = 1024
DW = 144
OWN = 5120
ACC = 5128

N_PTP = 50176
N_PLP = 10240
E1P = 65536
E2P = 163840


def _pad_rows(x, n):
    return jnp.pad(x, ((0, n - x.shape[0]),) + ((0, 0),) * (x.ndim - 1))


def _ln(x, g, b):
    mu = jnp.mean(x, axis=-1, keepdims=True)
    xc = x - mu
    v = jnp.mean(xc * xc, axis=-1, keepdims=True)
    return xc / jnp.sqrt(v + 1e-5) * g + b


def _embed_pt_body(mag, ct, freqs, w1, b1, ln1g, ln1b, w2, b2, ctab, olng,
                   olnb, ow, ob, o):
    xf = mag[...] * freqs[...] * (2.0 * np.pi)
    h = (jnp.cos(xf) @ w1[0:NFB, :] + jnp.sin(xf) @ w1[NFB:2 * NFB, :]
         + mag[...] * w1[2 * NFB:2 * NFB + 1, :] + b1[...])
    h = _ln(h, ln1g[...], ln1b[...])
    h = jax.nn.relu(h)
    x = h @ w2[...] + b2[...]
    oh = (ct[...] == lax.broadcasted_iota(jnp.int32, (R, ctab.shape[0]), 1))
    x = x + oh.astype(jnp.float32) @ ctab[...]
    x = _ln(x, olng[...], olnb[...])
    o[...] = jax.nn.relu(x) @ ow[...] + ob[...]


def _embed_pt(mag, ct, p):
    grid = N_PTP // R
    full = lambda a: pl.BlockSpec(a.shape, lambda i: (0,) * a.ndim)
    ctab = (p['type_pt_tab'][:, None, :] + p['side_pt_tab'][None, :, :]
            ).reshape(-1, HID)
    f = p['x_pt']
    args = (mag, ct, f['freqs'].reshape(1, NFB),
            f['w1'].reshape(2 * NFB + 1, HID),
            f['b1'].reshape(1, HID), f['ln1_g'].reshape(1, HID),
            f['ln1_b'].reshape(1, HID), f['w2'].reshape(HID, HID),
            f['b2'].reshape(1, HID), ctab, f['out_ln_g'].reshape(1, HID),
            f['out_ln_b'].reshape(1, HID), f['out_w'],
            f['out_b'].reshape(1, HID))
    specs = ([pl.BlockSpec((R, 1), lambda i: (i, 0)),
              pl.BlockSpec((R, 1), lambda i: (i, 0))]
             + [full(a) for a in args[2:]])
    return pl.pallas_call(
        _embed_pt_body, grid=(grid,), in_specs=specs,
        out_specs=pl.BlockSpec((R, HID), lambda i: (i, 0)),
        out_shape=jax.ShapeDtypeStruct((N_PTP, HID), jnp.float32),
    )(*args)


def _embed_pl_body(ct, ctab, olng, olnb, ow, ob, o):
    oh = (ct[...] == lax.broadcasted_iota(jnp.int32, (R, ctab.shape[0]), 1))
    x = oh.astype(jnp.float32) @ ctab[...]
    x = _ln(x, olng[...], olnb[...])
    o[...] = jax.nn.relu(x) @ ow[...] + ob[...]


def _embed_pl(ct, p):
    grid = N_PLP // R
    full = lambda a: pl.BlockSpec(a.shape, lambda i: (0,) * a.ndim)
    ctab = (p['type_pl_tab'][:, None, :] + p['int_pl_tab'][None, :, :]
            ).reshape(-1, HID)
    f = p['x_pl']
    args = (ct, ctab, f['out_ln_g'].reshape(1, HID),
            f['out_ln_b'].reshape(1, HID), f['out_w'],
            f['out_b'].reshape(1, HID))
    specs = ([pl.BlockSpec((R, 1), lambda i: (i, 0))]
             + [full(a) for a in args[1:]])
    return pl.pallas_call(
        _embed_pl_body, grid=(grid,), in_specs=specs,
        out_specs=pl.BlockSpec((R, HID), lambda i: (i, 0)),
        out_shape=jax.ShapeDtypeStruct((N_PLP, HID), jnp.float32),
    )(*args)


def _wrap_pi(a):
    w = a + np.pi
    w = w - (2.0 * np.pi) * jnp.floor(w / (2.0 * np.pi))
    return w - np.pi


def _edge_feat_body(has_cat, *refs):
    if has_cat:
        (sg, dg, tcat, cattab, freqs, w1, b1, ln1g, ln1b, w2, b2, olng, olnb,
         ow, ob, lnrg, lnrb, wkr, wvr, bvr, o) = refs
    else:
        (sg, dg, freqs, w1, b1, ln1g, ln1b, w2, b2, olng, olnb,
         ow, ob, lnrg, lnrb, wkr, wvr, bvr, o) = refs
    sx, sy, so = sg[:, 0:1], sg[:, 1:2], sg[:, 2:3]
    dx, dy, do_ = dg[:, 0:1], dg[:, 1:2], dg[:, 2:3]
    relx = sx - dx
    rely = sy - dy
    norm = jnp.sqrt(relx * relx + rely * rely)
    c = jnp.cos(do_)
    s = jnp.sin(do_)
    ang = jnp.arctan2(c * rely - s * relx, c * relx + s * rely)
    wrp = _wrap_pi(so - do_)
    rcols = (norm, ang, wrp)
    acc = None
    for d in range(3):
        rd = rcols[d]
        xf = rd * freqs[d:d + 1, :] * (2.0 * np.pi)
        h = (jnp.cos(xf) @ w1[d, 0:NFB, :] + jnp.sin(xf) @ w1[d, NFB:2 * NFB, :]
             + rd * w1[d, 2 * NFB:2 * NFB + 1, :] + b1[d:d + 1, :])
        h = _ln(h, ln1g[d:d + 1, :], ln1b[d:d + 1, :])
        h = jax.nn.relu(h)
        g = h @ w2[d] + b2[d:d + 1, :]
        acc = g if acc is None else acc + g
    if has_cat:
        oh = (tcat[...] == lax.broadcasted_iota(jnp.int32,
                                                (R, cattab.shape[0]), 1))
        acc = acc + oh.astype(jnp.float32) @ cattab[...]
    x = _ln(acc, olng[...], olnb[...])
    re = jax.nn.relu(x) @ ow[...] + ob[...]
    rn = _ln(re, lnrg[...], lnrb[...])
    o[...] = jnp.concatenate([rn @ wkr[...], rn @ wvr[...] + bvr[...]], axis=1)


def _edge_feat(sgeo, dgeo, tcat, fp, ap, cattab, ep):
    grid = ep // R
    full = lambda a: pl.BlockSpec(a.shape, lambda i: (0,) * a.ndim)
    has_cat = tcat is not None
    base = (fp['freqs'], fp['w1'], fp['b1'], fp['ln1_g'], fp['ln1_b'],
            fp['w2'], fp['b2'], fp['out_ln_g'].reshape(1, HID),
            fp['out_ln_b'].reshape(1, HID), fp['out_w'],
            fp['out_b'].reshape(1, HID), ap['ln_r_g'].reshape(1, HID),
            ap['ln_r_b'].reshape(1, HID), ap['wkr'], ap['wvr'],
            ap['bvr'].reshape(1, HID))
    if has_cat:
        args = (sgeo, dgeo, tcat, cattab) + base
        specs = [pl.BlockSpec((R, 16), lambda i: (i, 0)),
                 pl.BlockSpec((R, 16), lambda i: (i, 0)),
                 pl.BlockSpec((R, 1), lambda i: (i, 0)), full(cattab)]
    else:
        args = (sgeo, dgeo) + base
        specs = [pl.BlockSpec((R, 16), lambda i: (i, 0)),
                 pl.BlockSpec((R, 16), lambda i: (i, 0))]
    specs += [full(a) for a in base]
    return pl.pallas_call(
        functools.partial(_edge_feat_body, has_cat), grid=(grid,),
        in_specs=specs,
        out_specs=pl.BlockSpec((R, 2 * HID), lambda i: (i, 0)),
        out_shape=jax.ShapeDtypeStruct((ep, 2 * HID), jnp.float32),
    )(*args)


def _proj_kv_body(x, lng, lnb, wk, wv, bv, o):
    xs = _ln(x[...], lng[...], lnb[...])
    o[...] = jnp.concatenate([xs @ wk[...], xs @ wv[...] + bv[...]], axis=1)


def _proj_kv(x, lng, lnb, wk, wv, bv):
    n = x.shape[0]
    full = lambda a: pl.BlockSpec(a.shape, lambda i: (0,) * a.ndim)
    args = (x, lng.reshape(1, HID), lnb.reshape(1, HID), wk, wv,
            bv.reshape(1, HID))
    return pl.pallas_call(
        _proj_kv_body, grid=(n // R,),
        in_specs=[pl.BlockSpec((R, HID), lambda i: (i, 0))] +
                 [full(a) for a in args[1:]],
        out_specs=pl.BlockSpec((R, 2 * HID), lambda i: (i, 0)),
        out_shape=jax.ShapeDtypeStruct((n, 2 * HID), jnp.float32),
    )(*args)


def _proj_q_body(x, lng, lnb, wq, bq, o):
    xd = _ln(x[...], lng[...], lnb[...])
    o[...] = xd @ wq[...] + bq[...]


def _proj_q(x, lng, lnb, wq, bq):
    n = x.shape[0]
    full = lambda a: pl.BlockSpec(a.shape, lambda i: (0,) * a.ndim)
    args = (x, lng.reshape(1, HID), lnb.reshape(1, HID), wq,
            bq.reshape(1, HID))
    return pl.pallas_call(
        _proj_q_body, grid=(n // R,),
        in_specs=[pl.BlockSpec((R, HID), lambda i: (i, 0))] +
                 [full(a) for a in args[1:]],
        out_specs=pl.BlockSpec((R, HID), lambda i: (i, 0)),
        out_shape=jax.ShapeDtypeStruct((n, HID), jnp.float32),
    )(*args)


def _edge_attn_body(kvg, krvr, qg, pool, expand, o):
    ke = kvg[:, 0:HID] + krvr[:, 0:HID]
    ve = kvg[:, HID:2 * HID] + krvr[:, HID:2 * HID]
    sim = ((qg[...] * ke) @ pool[...]) * (1.0 / np.sqrt(D_HEAD))
    e = jnp.exp(sim)
    wv = ve * (e @ expand[...])
    ez = jnp.concatenate([e, jnp.zeros((e.shape[0], DW - HID - H), e.dtype)],
                         axis=1)
    o[...] = jnp.concatenate([wv, ez], axis=1)


def _edge_attn(kvg, krvr, qg, ep):
    pool = np.zeros((HID, H), np.float32)
    for h in range(H):
        pool[h * D_HEAD:(h + 1) * D_HEAD, h] = 1.0
    expand = jnp.asarray(np.ascontiguousarray(pool.T))
    pool = jnp.asarray(pool)
    full = lambda a: pl.BlockSpec(a.shape, lambda i: (0,) * a.ndim)
    return pl.pallas_call(
        _edge_attn_body, grid=(ep // R,),
        in_specs=[pl.BlockSpec((R, 2 * HID), lambda i: (i, 0)),
                  pl.BlockSpec((R, 2 * HID), lambda i: (i, 0)),
                  pl.BlockSpec((R, HID), lambda i: (i, 0)),
                  full(pool), full(expand)],
        out_specs=pl.BlockSpec((R, DW), lambda i: (i, 0)),
        out_shape=jax.ShapeDtypeStruct((ep, DW), jnp.float32),
    )(kvg, krvr, qg, pool, expand)


def _post_body(p0, xin, exp16, lng, lnb, wgt, wgb, bg, ws,
               bs, wo, bo, lnfg, lnfb, w1, w2, b2, o):
    ps = p0[...]
    num = ps[:, 0:HID]
    den = ps[:, HID:HID + 16] @ exp16[...]
    agg = num / (den + 1e-16)
    xd = _ln(xin[...], lng[...], lnb[...])
    g = jax.nn.sigmoid(xd @ wgt[...] + agg @ wgb[...] + bg[...])
    agg = agg + g * ((xd @ ws[...] + bs[...]) - agg)
    x = xin[...] + agg @ wo[...] + bo[...]
    hh = _ln(x, lnfg[...], lnfb[...])
    o[...] = x + jax.nn.relu(hh @ w1[...]) @ w2[...] + b2[...]


def _post(parts, xin, ap, ln_key):
    exp16 = np.zeros((16, HID), np.float32)
    for h in range(H):
        exp16[h, h * D_HEAD:(h + 1) * D_HEAD] = 1.0
    exp16 = jnp.asarray(exp16)
    full = lambda a: pl.BlockSpec(a.shape, lambda i: (0,) * a.ndim)
    consts = (exp16, ap[ln_key + '_g'].reshape(1, HID),
              ap[ln_key + '_b'].reshape(1, HID), ap['wg'][:HID, :],
              ap['wg'][HID:, :], ap['bg'].reshape(1, HID), ap['ws'],
              ap['bs'].reshape(1, HID), ap['wo'],
              ap['bo'].reshape(1, HID), ap['ln_ff_g'].reshape(1, HID),
              ap['ln_ff_b'].reshape(1, HID), ap['w1'], ap['w2'],
              ap['b2'].reshape(1, HID))
    return pl.pallas_call(
        _post_body, grid=(N_PLP // R,),
        in_specs=[pl.BlockSpec((R, DW), lambda i: (i, 0)),
                  pl.BlockSpec((R, HID), lambda i: (i, 0))] +
                 [full(a) for a in consts],
        out_specs=pl.BlockSpec((R, HID), lambda i: (i, 0)),
        out_shape=jax.ShapeDtypeStruct((N_PLP, HID), jnp.float32),
    )(parts, xin, *consts)


def _sc_gather2(ta, ia, tb, ib, ep, linear=False):
    da = ta.shape[1]
    db = tb.shape[1]
    ch = ep // (NW * CHUNK)
    ia = ia.reshape(NW, ch, CHUNK)
    ib = ib.reshape(NW, ch, CHUNK)
    mesh = plsc.VectorSubcoreMesh(core_axis_name="c", subcore_axis_name="s")

    assert ch % 2 == 0

    @functools.partial(
        pl.kernel, mesh=mesh,
        out_type=(jax.ShapeDtypeStruct((ep, da), jnp.float32),
                  jax.ShapeDtypeStruct((ep, db), jnp.float32)),
        scratch_types=[pltpu.VMEM((ch, CHUNK), jnp.int32),
                       pltpu.VMEM((ch, CHUNK), jnp.int32),
                       pltpu.VMEM((2, CHUNK, da), jnp.float32),
                       pltpu.VMEM((2, CHUNK, db), jnp.float32),
                       pltpu.SemaphoreType.DMA,
                       pltpu.SemaphoreType.DMA,
                       pltpu.SemaphoreType.DMA,
                       pltpu.SemaphoreType.DMA],
        compiler_params=pltpu.CompilerParams(use_tc_tiling_on_sc=not linear),
    )
    def k(ta_h, ia_h, tb_h, ib_h, oa_h, ob_h, ia_v, ib_v, ra_v, rb_v,
          sa0, sb0, sa1, sb1):
        wid = lax.axis_index("s") * NC + lax.axis_index("c")
        base0 = wid * ch * CHUNK
        pltpu.sync_copy(ia_h.at[wid], ia_v)
        pltpu.sync_copy(ib_h.at[wid], ib_v)
        sas = (sa0, sa1)
        sbs = (sb0, sb1)

        def issue(j, slot):
            pltpu.async_copy(ta_h.at[ia_v.at[j]], ra_v.at[slot], sas[slot])
            pltpu.async_copy(tb_h.at[ib_v.at[j]], rb_v.at[slot], sbs[slot])

        def drain(j, slot):
            pltpu.make_async_copy(ta_h.at[ia_v.at[j]], ra_v.at[slot],
                                  sas[slot]).wait()
            pltpu.make_async_copy(tb_h.at[ib_v.at[j]], rb_v.at[slot],
                                  sbs[slot]).wait()
            pltpu.sync_copy(ra_v.at[slot],
                            oa_h.at[pl.ds(base0 + j * CHUNK, CHUNK)])
            pltpu.sync_copy(rb_v.at[slot],
                            ob_h.at[pl.ds(base0 + j * CHUNK, CHUNK)])

        issue(0, 0)

        def body(t, carry):
            j0 = t * 2
            issue(j0 + 1, 1)
            drain(j0, 0)

            @pl.when(t + 1 < ch // 2)
            def _next():
                issue(j0 + 2, 0)

            drain(j0 + 1, 1)
            return carry

        lax.fori_loop(0, ch // 2, body, 0)

    return k(ta, ia, tb, ib)


def _sc_scatter_add(vals, idx, ep):
    chw = ep // (NS * CHUNK)
    idx = idx.reshape(NS, chw, CHUNK)
    zb = jnp.zeros((64, DW), jnp.float32)
    mesh = plsc.VectorSubcoreMesh(core_axis_name="c", subcore_axis_name="s")

    @functools.partial(
        pl.kernel, mesh=mesh,
        out_type=jax.ShapeDtypeStruct((N_PLP, DW), jnp.float32),
        scratch_types=[pltpu.VMEM((chw, CHUNK), jnp.int32),
                       pltpu.VMEM((chw, CHUNK), jnp.int32),
                       pltpu.VMEM((2, CHUNK, DW), jnp.float32),
                       pltpu.VMEM_SHARED((ACC, DW), jnp.float32),
                       pltpu.SemaphoreType.DMA,
                       pltpu.SemaphoreType.DMA],
        compiler_params=pltpu.CompilerParams(use_tc_tiling_on_sc=False),
    )
    def k(v_h, i_h, z_h, o_h, i_v, ia_v, v_v, acc, sv0, sv1):
        cid = lax.axis_index("c")
        sid = lax.axis_index("s")
        own0 = cid * OWN

        def zbody(t, carry):
            pltpu.sync_copy(z_h, acc.at[pl.ds(sid * (OWN // NS) + t * 64, 64)])
            return carry

        lax.fori_loop(0, OWN // NS // 64, zbody, 0)

        @pl.when(sid == 0)
        def _zero_tail():
            pltpu.sync_copy(z_h.at[pl.ds(0, ACC - OWN)],
                            acc.at[pl.ds(OWN, ACC - OWN)])

        pltpu.sync_copy(i_h.at[sid], i_v)

        def adj(j, carry):
            for l in range(CHUNK // 16):
                v = i_v[j, pl.ds(l * 16, 16)]
                ok = (v >= own0) & (v < own0 + OWN)
                ia_v[j, pl.ds(l * 16, 16)] = jnp.where(ok, v - own0, OWN)
            return carry

        lax.fori_loop(0, chw, adj, 0)
        plsc.subcore_barrier()

        base0 = sid * chw * CHUNK
        svs = (sv0, sv1)

        def issue(j, slot):
            pltpu.async_copy(v_h.at[pl.ds(base0 + j * CHUNK, CHUNK)],
                             v_v.at[slot], svs[slot])

        def drain(j, slot):
            pltpu.make_async_copy(v_h.at[pl.ds(base0 + j * CHUNK, CHUNK)],
                                  v_v.at[slot], svs[slot]).wait()
            pltpu.sync_copy(v_v.at[slot], acc.at[ia_v.at[j]], add=True)

        issue(0, 0)

        def body(t, carry):
            j0 = t * 2
            issue(j0 + 1, 1)
            drain(j0, 0)

            @pl.when(t + 1 < chw // 2)
            def _next():
                issue(j0 + 2, 0)

            drain(j0 + 1, 1)
            return carry

        lax.fori_loop(0, chw // 2, body, 0)
        plsc.subcore_barrier()
        rpt = OWN // NS
        pltpu.sync_copy(acc.at[pl.ds(sid * rpt, rpt)],
                        o_h.at[pl.ds(own0 + sid * rpt, rpt)])

    return k(vals, idx, zb)


def kernel(pos_pt, orient_pt, magnitude_pt, type_pt, side_pt, pos_pl,
           orient_pl, type_pl, int_pl, edge_index_pt2pl, edge_index_pl2pl,
           type_pl2pl, params):
    p = params
    f32 = jnp.float32

    mag = _pad_rows(magnitude_pt[:, None].astype(f32), N_PTP)
    ct_pt = _pad_rows((type_pt.astype(jnp.int32) * 3
                       + side_pt.astype(jnp.int32))[:, None], N_PTP)
    ct_pl = _pad_rows((type_pl.astype(jnp.int32) * 3
                       + int_pl.astype(jnp.int32))[:, None], N_PLP)
    pt_params = {'type_pt_tab': p['type_pt'], 'side_pt_tab': p['side_pt'],
                 'x_pt': p['x_pt']}
    pl_params = {'type_pl_tab': p['type_pl'], 'int_pl_tab': p['int_pl'],
                 'x_pl': p['x_pl']}
    x_pt = _embed_pt(mag, ct_pt, pt_params)
    x_pl = _embed_pl(ct_pl, pl_params)

    geo_pt = jnp.concatenate(
        [pos_pt.astype(f32), orient_pt[:, None].astype(f32),
         jnp.zeros((N_PT, 13), f32)], axis=1)
    geo_pl = jnp.concatenate(
        [pos_pl.astype(f32), orient_pl[:, None].astype(f32),
         jnp.zeros((N_PL, 13), f32)], axis=1)
    s1 = edge_index_pt2pl[0].astype(jnp.int32)
    d1 = edge_index_pt2pl[1].astype(jnp.int32)
    s2 = edge_index_pl2pl[0].astype(jnp.int32)
    d2 = edge_index_pl2pl[1].astype(jnp.int32)
    pad_i = lambda a, n, v: jnp.pad(a, (0, n - a.shape[0]), constant_values=v)
    s1p = pad_i(s1, E1P, 0)
    d1p = pad_i(d1, E1P, 0)
    d1s = pad_i(d1, E1P, N_PLP - 1)
    s2p = pad_i(s2, E2P, 0)
    d2p = pad_i(d2, E2P, 0)
    d2s = pad_i(d2, E2P, N_PLP - 1)
    tc2 = _pad_rows(type_pl2pl.astype(jnp.int32)[:, None], E2P)

    sg1, dg1 = _sc_gather2(geo_pt, s1p, geo_pl, d1p, E1P, linear=True)
    sg2, dg2 = _sc_gather2(geo_pl, s2p, geo_pl, d2p, E2P, linear=True)

    a1 = p['pt2pl'][0]
    a2 = p['pl2pl'][0]
    krvr1 = _edge_feat(sg1, dg1, None, p['r_pt2pl'], a1, None, E1P)
    krvr2 = _edge_feat(sg2, dg2, tc2, p['r_pl2pl'], a2, p['type_pl2pl'], E2P)

    kv1 = _proj_kv(x_pt, a1['ln_s_g'], a1['ln_s_b'], a1['wk'], a1['wv'],
                   a1['bv'])
    q1 = _proj_q(x_pl, a1['ln_d_g'], a1['ln_d_b'], a1['wq'], a1['bq'])
    kvg1, qg1 = _sc_gather2(kv1, s1p, q1, d1p, E1P)
    ew1 = _edge_attn(kvg1, krvr1, qg1, E1P)
    parts1 = _sc_scatter_add(ew1, d1s, E1P)
    x_pl = _post(parts1, x_pl, a1, 'ln_d')

    kv2 = _proj_kv(x_pl, a2['ln_s_g'], a2['ln_s_b'], a2['wk'], a2['wv'],
                   a2['bv'])
    q2 = _proj_q(x_pl, a2['ln_s_g'], a2['ln_s_b'], a2['wq'], a2['bq'])
    kvg2, qg2 = _sc_gather2(kv2, s2p, q2, d2p, E2P)
    ew2 = _edge_attn(kvg2, krvr2, qg2, E2P)
    parts2 = _sc_scatter_add(ew2, d2s, E2P)
    x_pl = _post(parts2, x_pl, a2, 'ln_s')

    return x_pl[:N_PL]


# speedup vs baseline: 1.0801x; 1.0751x over previous
import functools

import jax
import jax.numpy as jnp
import numpy as np
from jax import lax
from jax.experimental import pallas as pl
from jax.experimental.pallas import tpu as pltpu
from jax.experimental.pallas import tpu_sc as plsc

H = 8
D_HEAD = 16
HID = 128
NFB = 64
N_PT = 50000
N_PL = 10000
E1 = 50000
E2 = 160000

NC = 2
NS = 16
NW = NC * NS
CHUNK = 128

R = 1024
DW = 144
OWN = 5120
ACC = 5128

N_PTP = 50176
N_PLP = 10240
E1P = 65536
E2P = 163840


def _pad_rows(x, n):
    return jnp.pad(x, ((0, n - x.shape[0]),) + ((0, 0),) * (x.ndim - 1))


def _ln(x, g, b):
    d = x.shape[-1]
    j = jnp.full((d, d), 1.0 / d, x.dtype)
    mu = x @ j
    m2 = (x * x) @ j
    v = m2 - mu * mu
    return (x - mu) * lax.rsqrt(v + 1e-5) * g + b


def _embed_pt_body(mag, ct, f2p8, w1, w1x8, b1, ln1g, ln1b, w2, b2, ctab,
                   olng, olnb, ow, ob, o):
    xf = mag[...] @ f2p8[...]
    h = (jnp.cos(xf) @ w1[0:NFB, :] + jnp.sin(xf) @ w1[NFB:2 * NFB, :]
         + mag[...] @ w1x8[...] + b1[...])
    h = _ln(h, ln1g[...], ln1b[...])
    h = jax.nn.relu(h)
    x = h @ w2[...] + b2[...]
    oh = (ct[...] == lax.broadcasted_iota(jnp.int32, (R, ctab.shape[0]), 1))
    x = x + oh.astype(jnp.float32) @ ctab[...]
    x = _ln(x, olng[...], olnb[...])
    o[...] = jax.nn.relu(x) @ ow[...] + ob[...]


def _embed_pt(mag, ct, p):
    grid = N_PTP // R
    full = lambda a: pl.BlockSpec(a.shape, lambda i: (0,) * a.ndim)
    ctab = (p['type_pt_tab'][:, None, :] + p['side_pt_tab'][None, :, :]
            ).reshape(-1, HID)
    f = p['x_pt']
    f2p8 = jnp.zeros((8, NFB), jnp.float32).at[0].set(
        f['freqs'][0] * (2.0 * np.pi))
    w1x8 = jnp.zeros((8, HID), jnp.float32).at[0].set(f['w1'][0, 2 * NFB])
    args = (mag, ct, f2p8,
            f['w1'].reshape(2 * NFB + 1, HID), w1x8,
            f['b1'].reshape(1, HID), f['ln1_g'].reshape(1, HID),
            f['ln1_b'].reshape(1, HID), f['w2'].reshape(HID, HID),
            f['b2'].reshape(1, HID), ctab, f['out_ln_g'].reshape(1, HID),
            f['out_ln_b'].reshape(1, HID), f['out_w'],
            f['out_b'].reshape(1, HID))
    specs = ([pl.BlockSpec((R, 8), lambda i: (i, 0)),
              pl.BlockSpec((R, 1), lambda i: (i, 0))]
             + [full(a) for a in args[2:]])
    return pl.pallas_call(
        _embed_pt_body, grid=(grid,), in_specs=specs,
        out_specs=pl.BlockSpec((R, HID), lambda i: (i, 0)),
        out_shape=jax.ShapeDtypeStruct((N_PTP, HID), jnp.float32),
    )(*args)


def _embed_pl_body(ct, ctab, olng, olnb, ow, ob, o):
    oh = (ct[...] == lax.broadcasted_iota(jnp.int32, (R, ctab.shape[0]), 1))
    x = oh.astype(jnp.float32) @ ctab[...]
    x = _ln(x, olng[...], olnb[...])
    o[...] = jax.nn.relu(x) @ ow[...] + ob[...]


def _embed_pl(ct, p):
    grid = N_PLP // R
    full = lambda a: pl.BlockSpec(a.shape, lambda i: (0,) * a.ndim)
    ctab = (p['type_pl_tab'][:, None, :] + p['int_pl_tab'][None, :, :]
            ).reshape(-1, HID)
    f = p['x_pl']
    args = (ct, ctab, f['out_ln_g'].reshape(1, HID),
            f['out_ln_b'].reshape(1, HID), f['out_w'],
            f['out_b'].reshape(1, HID))
    specs = ([pl.BlockSpec((R, 1), lambda i: (i, 0))]
             + [full(a) for a in args[1:]])
    return pl.pallas_call(
        _embed_pl_body, grid=(grid,), in_specs=specs,
        out_specs=pl.BlockSpec((R, HID), lambda i: (i, 0)),
        out_shape=jax.ShapeDtypeStruct((N_PLP, HID), jnp.float32),
    )(*args)


def _wrap_pi(a):
    w = a + np.pi
    w = w - (2.0 * np.pi) * jnp.floor(w / (2.0 * np.pi))
    return w - np.pi


def _ln_j(x, g, b, j):
    mu = x @ j
    m2 = (x * x) @ j
    v = m2 - mu * mu
    return (x - mu) * lax.rsqrt(v + 1e-5) * g + b


def _edge_feat_body(has_cat, *refs):
    if has_cat:
        (sg, dg, tcat, cattab, f2p, w1cs, w1x, b1c, ln1g, ln1b, jg, w2s, b2s,
         olng, olnb, ow, ob, lnrg, lnrb, wkr, wvr, bvr, o) = refs
    else:
        (sg, dg, f2p, w1cs, w1x, b1c, ln1g, ln1b, jg, w2s, b2s,
         olng, olnb, ow, ob, lnrg, lnrb, wkr, wvr, bvr, o) = refs
    sx, sy, so = sg[:, 0:1], sg[:, 1:2], sg[:, 2:3]
    dx, dy, do_ = dg[:, 0:1], dg[:, 1:2], dg[:, 2:3]
    relx = sx - dx
    rely = sy - dy
    norm = jnp.sqrt(relx * relx + rely * rely)
    c = jnp.cos(do_)
    s = jnp.sin(do_)
    ang = jnp.arctan2(c * rely - s * relx, c * relx + s * rely)
    wrp = _wrap_pi(so - do_)
    rcat = jnp.concatenate([norm, ang, wrp], axis=1)
    xf = rcat @ f2p[...]
    fcat = jnp.concatenate([jnp.cos(xf), jnp.sin(xf)], axis=1)
    h = fcat @ w1cs[...] + rcat @ w1x[...] + b1c[...]
    h = _ln_j(h, ln1g[...], ln1b[...], jg[...])
    h = jax.nn.relu(h)
    acc = h @ w2s[...] + b2s[...]
    if has_cat:
        oh = (tcat[...] == lax.broadcasted_iota(jnp.int32,
                                                (R, cattab.shape[0]), 1))
        acc = acc + oh.astype(jnp.float32) @ cattab[...]
    x = _ln(acc, olng[...], olnb[...])
    re = jax.nn.relu(x) @ ow[...] + ob[...]
    rn = _ln(re, lnrg[...], lnrb[...])
    o[...] = jnp.concatenate([rn @ wkr[...], rn @ wvr[...] + bvr[...]], axis=1)


def _edge_feat(sgeo, dgeo, tcat, fp, ap, cattab, ep):
    grid = ep // R
    full = lambda a: pl.BlockSpec(a.shape, lambda i: (0,) * a.ndim)
    has_cat = tcat is not None
    f2p = jnp.zeros((3, 3 * NFB), jnp.float32)
    w1cs = jnp.zeros((6 * NFB, 3 * HID), jnp.float32)
    w1x = jnp.zeros((3, 3 * HID), jnp.float32)
    for d in range(3):
        f2p = f2p.at[d, d * NFB:(d + 1) * NFB].set(
            fp['freqs'][d] * (2.0 * np.pi))
        w1cs = w1cs.at[d * NFB:(d + 1) * NFB,
                       d * HID:(d + 1) * HID].set(fp['w1'][d, 0:NFB])
        w1cs = w1cs.at[3 * NFB + d * NFB:3 * NFB + (d + 1) * NFB,
                       d * HID:(d + 1) * HID].set(fp['w1'][d, NFB:2 * NFB])
        w1x = w1x.at[d, d * HID:(d + 1) * HID].set(fp['w1'][d, 2 * NFB])
    jg = jnp.asarray(np.kron(np.eye(3, dtype=np.float32),
                             np.full((HID, HID), 1.0 / HID, np.float32)))
    base = (f2p, w1cs, w1x, fp['b1'].reshape(1, 3 * HID),
            fp['ln1_g'].reshape(1, 3 * HID), fp['ln1_b'].reshape(1, 3 * HID),
            jg, fp['w2'].reshape(3 * HID, HID),
            jnp.sum(fp['b2'], axis=0, keepdims=True),
            fp['out_ln_g'].reshape(1, HID),
            fp['out_ln_b'].reshape(1, HID), fp['out_w'],
            fp['out_b'].reshape(1, HID), ap['ln_r_g'].reshape(1, HID),
            ap['ln_r_b'].reshape(1, HID), ap['wkr'], ap['wvr'],
            ap['bvr'].reshape(1, HID))
    if has_cat:
        args = (sgeo, dgeo, tcat, cattab) + base
        specs = [pl.BlockSpec((R, 16), lambda i: (i, 0)),
                 pl.BlockSpec((R, 16), lambda i: (i, 0)),
                 pl.BlockSpec((R, 1), lambda i: (i, 0)), full(cattab)]
    else:
        args = (sgeo, dgeo) + base
        specs = [pl.BlockSpec((R, 16), lambda i: (i, 0)),
                 pl.BlockSpec((R, 16), lambda i: (i, 0))]
    specs += [full(a) for a in base]
    return pl.pallas_call(
        functools.partial(_edge_feat_body, has_cat), grid=(grid,),
        in_specs=specs,
        out_specs=pl.BlockSpec((R, 2 * HID), lambda i: (i, 0)),
        out_shape=jax.ShapeDtypeStruct((ep, 2 * HID), jnp.float32),
    )(*args)


def _proj_kv_body(x, lng, lnb, wk, wv, bv, o):
    xs = _ln(x[...], lng[...], lnb[...])
    o[...] = jnp.concatenate([xs @ wk[...], xs @ wv[...] + bv[...]], axis=1)


def _proj_kv(x, lng, lnb, wk, wv, bv):
    n = x.shape[0]
    full = lambda a: pl.BlockSpec(a.shape, lambda i: (0,) * a.ndim)
    args = (x, lng.reshape(1, HID), lnb.reshape(1, HID), wk, wv,
            bv.reshape(1, HID))
    return pl.pallas_call(
        _proj_kv_body, grid=(n // R,),
        in_specs=[pl.BlockSpec((R, HID), lambda i: (i, 0))] +
                 [full(a) for a in args[1:]],
        out_specs=pl.BlockSpec((R, 2 * HID), lambda i: (i, 0)),
        out_shape=jax.ShapeDtypeStruct((n, 2 * HID), jnp.float32),
    )(*args)


def _proj_q_body(x, lng, lnb, wq, bq, o):
    xd = _ln(x[...], lng[...], lnb[...])
    o[...] = xd @ wq[...] + bq[...]


def _proj_q(x, lng, lnb, wq, bq):
    n = x.shape[0]
    full = lambda a: pl.BlockSpec(a.shape, lambda i: (0,) * a.ndim)
    args = (x, lng.reshape(1, HID), lnb.reshape(1, HID), wq,
            bq.reshape(1, HID))
    return pl.pallas_call(
        _proj_q_body, grid=(n // R,),
        in_specs=[pl.BlockSpec((R, HID), lambda i: (i, 0))] +
                 [full(a) for a in args[1:]],
        out_specs=pl.BlockSpec((R, HID), lambda i: (i, 0)),
        out_shape=jax.ShapeDtypeStruct((n, HID), jnp.float32),
    )(*args)


def _edge_attn_body(kvg, krvr, qg, pool, expand, o):
    ke = kvg[:, 0:HID] + krvr[:, 0:HID]
    ve = kvg[:, HID:2 * HID] + krvr[:, HID:2 * HID]
    sim = ((qg[...] * ke) @ pool[...]) * (1.0 / np.sqrt(D_HEAD))
    e = jnp.exp(sim)
    wv = ve * (e @ expand[...])
    ez = jnp.concatenate([e, jnp.zeros((e.shape[0], DW - HID - H), e.dtype)],
                         axis=1)
    o[...] = jnp.concatenate([wv, ez], axis=1)


def _edge_attn(kvg, krvr, qg, ep):
    pool = np.zeros((HID, H), np.float32)
    for h in range(H):
        pool[h * D_HEAD:(h + 1) * D_HEAD, h] = 1.0
    expand = jnp.asarray(np.ascontiguousarray(pool.T))
    pool = jnp.asarray(pool)
    full = lambda a: pl.BlockSpec(a.shape, lambda i: (0,) * a.ndim)
    return pl.pallas_call(
        _edge_attn_body, grid=(ep // R,),
        in_specs=[pl.BlockSpec((R, 2 * HID), lambda i: (i, 0)),
                  pl.BlockSpec((R, 2 * HID), lambda i: (i, 0)),
                  pl.BlockSpec((R, HID), lambda i: (i, 0)),
                  full(pool), full(expand)],
        out_specs=pl.BlockSpec((R, DW), lambda i: (i, 0)),
        out_shape=jax.ShapeDtypeStruct((ep, DW), jnp.float32),
    )(kvg, krvr, qg, pool, expand)


def _post_body(p0, xin, exp16, lng, lnb, wgt, wgb, bg, ws,
               bs, wo, bo, lnfg, lnfb, w1, w2, b2, o):
    ps = p0[...]
    num = ps[:, 0:HID]
    den = ps[:, HID:HID + 16] @ exp16[...]
    agg = num / (den + 1e-16)
    xd = _ln(xin[...], lng[...], lnb[...])
    g = jax.nn.sigmoid(xd @ wgt[...] + agg @ wgb[...] + bg[...])
    agg = agg + g * ((xd @ ws[...] + bs[...]) - agg)
    x = xin[...] + agg @ wo[...] + bo[...]
    hh = _ln(x, lnfg[...], lnfb[...])
    o[...] = x + jax.nn.relu(hh @ w1[...]) @ w2[...] + b2[...]


def _post(parts, xin, ap, ln_key):
    exp16 = np.zeros((16, HID), np.float32)
    for h in range(H):
        exp16[h, h * D_HEAD:(h + 1) * D_HEAD] = 1.0
    exp16 = jnp.asarray(exp16)
    full = lambda a: pl.BlockSpec(a.shape, lambda i: (0,) * a.ndim)
    consts = (exp16, ap[ln_key + '_g'].reshape(1, HID),
              ap[ln_key + '_b'].reshape(1, HID), ap['wg'][:HID, :],
              ap['wg'][HID:, :], ap['bg'].reshape(1, HID), ap['ws'],
              ap['bs'].reshape(1, HID), ap['wo'],
              ap['bo'].reshape(1, HID), ap['ln_ff_g'].reshape(1, HID),
              ap['ln_ff_b'].reshape(1, HID), ap['w1'], ap['w2'],
              ap['b2'].reshape(1, HID))
    return pl.pallas_call(
        _post_body, grid=(N_PLP // R,),
        in_specs=[pl.BlockSpec((R, DW), lambda i: (i, 0)),
                  pl.BlockSpec((R, HID), lambda i: (i, 0))] +
                 [full(a) for a in consts],
        out_specs=pl.BlockSpec((R, HID), lambda i: (i, 0)),
        out_shape=jax.ShapeDtypeStruct((N_PLP, HID), jnp.float32),
    )(parts, xin, *consts)


def _sc_gather2(ta, ia, tb, ib, ep, linear=False):
    da = ta.shape[1]
    db = tb.shape[1]
    ch = ep // (NW * CHUNK)
    ia = ia.reshape(NW, ch, CHUNK)
    ib = ib.reshape(NW, ch, CHUNK)
    mesh = plsc.VectorSubcoreMesh(core_axis_name="c", subcore_axis_name="s")

    assert ch % 2 == 0

    @functools.partial(
        pl.kernel, mesh=mesh,
        out_type=(jax.ShapeDtypeStruct((ep, da), jnp.float32),
                  jax.ShapeDtypeStruct((ep, db), jnp.float32)),
        scratch_types=[pltpu.VMEM((ch, CHUNK), jnp.int32),
                       pltpu.VMEM((ch, CHUNK), jnp.int32),
                       pltpu.VMEM((2, CHUNK, da), jnp.float32),
                       pltpu.VMEM((2, CHUNK, db), jnp.float32),
                       pltpu.SemaphoreType.DMA,
                       pltpu.SemaphoreType.DMA,
                       pltpu.SemaphoreType.DMA,
                       pltpu.SemaphoreType.DMA],
        compiler_params=pltpu.CompilerParams(use_tc_tiling_on_sc=not linear),
    )
    def k(ta_h, ia_h, tb_h, ib_h, oa_h, ob_h, ia_v, ib_v, ra_v, rb_v,
          sa0, sb0, sa1, sb1):
        wid = lax.axis_index("s") * NC + lax.axis_index("c")
        base0 = wid * ch * CHUNK
        pltpu.sync_copy(ia_h.at[wid], ia_v)
        pltpu.sync_copy(ib_h.at[wid], ib_v)
        sas = (sa0, sa1)
        sbs = (sb0, sb1)

        def issue(j, slot):
            pltpu.async_copy(ta_h.at[ia_v.at[j]], ra_v.at[slot], sas[slot])
            pltpu.async_copy(tb_h.at[ib_v.at[j]], rb_v.at[slot], sbs[slot])

        def drain(j, slot):
            pltpu.make_async_copy(ta_h.at[ia_v.at[j]], ra_v.at[slot],
                                  sas[slot]).wait()
            pltpu.make_async_copy(tb_h.at[ib_v.at[j]], rb_v.at[slot],
                                  sbs[slot]).wait()
            pltpu.sync_copy(ra_v.at[slot],
                            oa_h.at[pl.ds(base0 + j * CHUNK, CHUNK)])
            pltpu.sync_copy(rb_v.at[slot],
                            ob_h.at[pl.ds(base0 + j * CHUNK, CHUNK)])

        issue(0, 0)

        def body(t, carry):
            j0 = t * 2
            issue(j0 + 1, 1)
            drain(j0, 0)

            @pl.when(t + 1 < ch // 2)
            def _next():
                issue(j0 + 2, 0)

            drain(j0 + 1, 1)
            return carry

        lax.fori_loop(0, ch // 2, body, 0)

    return k(ta, ia, tb, ib)


def _sc_scatter_add(vals, idx, ep):
    chw = ep // (NS * CHUNK)
    idx = idx.reshape(NS, chw, CHUNK)
    zb = jnp.zeros((64, DW), jnp.float32)
    mesh = plsc.VectorSubcoreMesh(core_axis_name="c", subcore_axis_name="s")

    @functools.partial(
        pl.kernel, mesh=mesh,
        out_type=jax.ShapeDtypeStruct((N_PLP, DW), jnp.float32),
        scratch_types=[pltpu.VMEM((chw, CHUNK), jnp.int32),
                       pltpu.VMEM((chw, CHUNK), jnp.int32),
                       pltpu.VMEM((2, CHUNK, DW), jnp.float32),
                       pltpu.VMEM_SHARED((ACC, DW), jnp.float32),
                       pltpu.SemaphoreType.DMA,
                       pltpu.SemaphoreType.DMA],
        compiler_params=pltpu.CompilerParams(use_tc_tiling_on_sc=False),
    )
    def k(v_h, i_h, z_h, o_h, i_v, ia_v, v_v, acc, sv0, sv1):
        cid = lax.axis_index("c")
        sid = lax.axis_index("s")
        own0 = cid * OWN

        def zbody(t, carry):
            pltpu.sync_copy(z_h, acc.at[pl.ds(sid * (OWN // NS) + t * 64, 64)])
            return carry

        lax.fori_loop(0, OWN // NS // 64, zbody, 0)

        @pl.when(sid == 0)
        def _zero_tail():
            pltpu.sync_copy(z_h.at[pl.ds(0, ACC - OWN)],
                            acc.at[pl.ds(OWN, ACC - OWN)])

        pltpu.sync_copy(i_h.at[sid], i_v)

        def adj(j, carry):
            for l in range(CHUNK // 16):
                v = i_v[j, pl.ds(l * 16, 16)]
                ok = (v >= own0) & (v < own0 + OWN)
                ia_v[j, pl.ds(l * 16, 16)] = jnp.where(ok, v - own0, OWN)
            return carry

        lax.fori_loop(0, chw, adj, 0)
        plsc.subcore_barrier()

        base0 = sid * chw * CHUNK
        svs = (sv0, sv1)

        def issue(j, slot):
            pltpu.async_copy(v_h.at[pl.ds(base0 + j * CHUNK, CHUNK)],
                             v_v.at[slot], svs[slot])

        def drain(j, slot):
            pltpu.make_async_copy(v_h.at[pl.ds(base0 + j * CHUNK, CHUNK)],
                                  v_v.at[slot], svs[slot]).wait()
            pltpu.sync_copy(v_v.at[slot], acc.at[ia_v.at[j]], add=True)

        issue(0, 0)

        def body(t, carry):
            j0 = t * 2
            issue(j0 + 1, 1)
            drain(j0, 0)

            @pl.when(t + 1 < chw // 2)
            def _next():
                issue(j0 + 2, 0)

            drain(j0 + 1, 1)
            return carry

        lax.fori_loop(0, chw // 2, body, 0)
        plsc.subcore_barrier()
        rpt = OWN // NS
        pltpu.sync_copy(acc.at[pl.ds(sid * rpt, rpt)],
                        o_h.at[pl.ds(own0 + sid * rpt, rpt)])

    return k(vals, idx, zb)


def kernel(pos_pt, orient_pt, magnitude_pt, type_pt, side_pt, pos_pl,
           orient_pl, type_pl, int_pl, edge_index_pt2pl, edge_index_pl2pl,
           type_pl2pl, params):
    p = params
    f32 = jnp.float32

    mag = jnp.pad(magnitude_pt[:, None].astype(f32),
                  ((0, N_PTP - N_PT), (0, 7)))
    ct_pt = _pad_rows((type_pt.astype(jnp.int32) * 3
                       + side_pt.astype(jnp.int32))[:, None], N_PTP)
    ct_pl = _pad_rows((type_pl.astype(jnp.int32) * 3
                       + int_pl.astype(jnp.int32))[:, None], N_PLP)
    pt_params = {'type_pt_tab': p['type_pt'], 'side_pt_tab': p['side_pt'],
                 'x_pt': p['x_pt']}
    pl_params = {'type_pl_tab': p['type_pl'], 'int_pl_tab': p['int_pl'],
                 'x_pl': p['x_pl']}
    x_pt = _embed_pt(mag, ct_pt, pt_params)
    x_pl = _embed_pl(ct_pl, pl_params)

    geo_pt = jnp.concatenate(
        [pos_pt.astype(f32), orient_pt[:, None].astype(f32),
         jnp.zeros((N_PT, 13), f32)], axis=1)
    geo_pl = jnp.concatenate(
        [pos_pl.astype(f32), orient_pl[:, None].astype(f32),
         jnp.zeros((N_PL, 13), f32)], axis=1)
    s1 = edge_index_pt2pl[0].astype(jnp.int32)
    d1 = edge_index_pt2pl[1].astype(jnp.int32)
    s2 = edge_index_pl2pl[0].astype(jnp.int32)
    d2 = edge_index_pl2pl[1].astype(jnp.int32)
    pad_i = lambda a, n, v: jnp.pad(a, (0, n - a.shape[0]), constant_values=v)
    s1p = pad_i(s1, E1P, 0)
    d1p = pad_i(d1, E1P, 0)
    d1s = pad_i(d1, E1P, N_PLP - 1)
    s2p = pad_i(s2, E2P, 0)
    d2p = pad_i(d2, E2P, 0)
    d2s = pad_i(d2, E2P, N_PLP - 1)
    tc2 = _pad_rows(type_pl2pl.astype(jnp.int32)[:, None], E2P)

    sg1, dg1 = _sc_gather2(geo_pt, s1p, geo_pl, d1p, E1P, linear=True)
    sg2, dg2 = _sc_gather2(geo_pl, s2p, geo_pl, d2p, E2P, linear=True)

    a1 = p['pt2pl'][0]
    a2 = p['pl2pl'][0]
    krvr1 = _edge_feat(sg1, dg1, None, p['r_pt2pl'], a1, None, E1P)
    krvr2 = _edge_feat(sg2, dg2, tc2, p['r_pl2pl'], a2, p['type_pl2pl'], E2P)

    kv1 = _proj_kv(x_pt, a1['ln_s_g'], a1['ln_s_b'], a1['wk'], a1['wv'],
                   a1['bv'])
    q1 = _proj_q(x_pl, a1['ln_d_g'], a1['ln_d_b'], a1['wq'], a1['bq'])
    kvg1, qg1 = _sc_gather2(kv1, s1p, q1, d1p, E1P)
    ew1 = _edge_attn(kvg1, krvr1, qg1, E1P)
    parts1 = _sc_scatter_add(ew1, d1s, E1P)
    x_pl = _post(parts1, x_pl, a1, 'ln_d')

    kv2 = _proj_kv(x_pl, a2['ln_s_g'], a2['ln_s_b'], a2['wk'], a2['wv'],
                   a2['bv'])
    q2 = _proj_q(x_pl, a2['ln_s_g'], a2['ln_s_b'], a2['wq'], a2['bq'])
    kvg2, qg2 = _sc_gather2(kv2, s2p, q2, d2p, E2P)
    ew2 = _edge_attn(kvg2, krvr2, qg2, E2P)
    parts2 = _sc_scatter_add(ew2, d2s, E2P)
    x_pl = _post(parts2, x_pl, a2, 'ln_s')

    return x_pl[:N_PL]
